# Initial kernel scaffold; baseline (speedup 1.0000x reference)
import jax
import jax.numpy as jnp
from jax.experimental import pallas as pl


def kernel(features, edge_index, Wq1, bq1, Wk1, bk1, Wv1, bv1, Ws1, bs1, Wq2, bq2, Wk2, bk2, Wv2, bv2, Ws2, bs2):
    raise NotImplementedError("write your pallas kernel here")


# speedup vs baseline: 6.5948x; 6.5948x over previous
import functools

import jax
import jax.numpy as jnp
from jax import lax
from jax.experimental import pallas as pl
from jax.experimental.pallas import tpu as pltpu
from jax.experimental.pallas import tpu_sc as plsc

N_NODES = 10000
N_EDGES = 320000
NC = 2
NS = 16
NW = NC * NS
CH = 128
NCHUNKS = N_EDGES // CH
NP = 10240
RPTB = NP // NS
WROWS = 80

_SC_MESH = dict(core_axis_name="c", subcore_axis_name="s", num_cores=NC,
                num_subcores=NS)
_SC_PARAMS = pltpu.CompilerParams(needs_layout_passes=False)

f32 = jnp.float32
i32 = jnp.int32


def _proj(x, w, b, rows=2000):
    m, kdim = x.shape
    nout = w.shape[1]

    def body(x_ref, w_ref, b_ref, o_ref):
        o_ref[...] = jnp.dot(x_ref[...], w_ref[...],
                             preferred_element_type=f32) + b_ref[...]

    return pl.pallas_call(
        body,
        grid=(m // rows,),
        in_specs=[pl.BlockSpec((rows, kdim), lambda i: (i, 0)),
                  pl.BlockSpec((kdim, nout), lambda i: (0, 0)),
                  pl.BlockSpec((1, nout), lambda i: (0, 0))],
        out_specs=pl.BlockSpec((rows, nout), lambda i: (i, 0)),
        out_shape=jax.ShapeDtypeStruct((m, nout), f32),
    )(x, w, b)


def _agg_specs(rows, d, extra):
    return [pl.BlockSpec((rows, 128), lambda i: (i, 0)),
            pl.BlockSpec((rows, 128), lambda i: (i, 0)),
            pl.BlockSpec((rows, 1), lambda i: (i, 0)),
            pl.BlockSpec((rows, 1), lambda i: (i, 0)),
            pl.BlockSpec((rows, d), lambda i: (i, 0))] + extra


def _norm_agg(a0_ref, a1_ref, w0_ref, w1_ref, s_ref, concat_halves):
    if concat_halves:
        agg = jnp.concatenate([a0_ref[...], a1_ref[...]], axis=1)
    else:
        agg = a0_ref[...] + a1_ref[...]
    ws = w0_ref[...] + w1_ref[...]
    return jnp.maximum(agg / (ws + 1e-16) + s_ref[...], 0.0)


def _combine_proj(a0, a1, w0, w1, s, wmat, b, concat_halves, rows=2000):
    m, d = s.shape
    nout = wmat.shape[1]

    def body(a0_ref, a1_ref, w0_ref, w1_ref, s_ref, w_ref, b_ref, o_ref):
        h = _norm_agg(a0_ref, a1_ref, w0_ref, w1_ref, s_ref, concat_halves)
        o_ref[...] = jnp.dot(h, w_ref[...], preferred_element_type=f32) \
            + b_ref[...]

    return pl.pallas_call(
        body,
        grid=(m // rows,),
        in_specs=_agg_specs(rows, d, [
            pl.BlockSpec((d, nout), lambda i: (0, 0)),
            pl.BlockSpec((1, nout), lambda i: (0, 0))]),
        out_specs=pl.BlockSpec((rows, nout), lambda i: (i, 0)),
        out_shape=jax.ShapeDtypeStruct((m, nout), f32),
    )(a0, a1, w0, w1, s, wmat, b)


def _combine_final(a0, a1, w0, w1, s, concat_halves, rows=2000):
    m, d = s.shape

    def body(a0_ref, a1_ref, w0_ref, w1_ref, s_ref, o_ref):
        x = _norm_agg(a0_ref, a1_ref, w0_ref, w1_ref, s_ref, concat_halves)
        mx = jnp.max(x, axis=1, keepdims=True)
        lse = jnp.log(jnp.sum(jnp.exp(x - mx), axis=1, keepdims=True)) + mx
        o_ref[...] = x - lse

    return pl.pallas_call(
        body,
        grid=(m // rows,),
        in_specs=_agg_specs(rows, d, []),
        out_specs=pl.BlockSpec((rows, d), lambda i: (i, 0)),
        out_shape=jax.ShapeDtypeStruct((m, d), f32),
    )(a0, a1, w0, w1, s)


def _phase_w(q, k, dst, src, z, d):
    dc = d // 16
    scale = 1.0 / (d ** 0.5)
    mesh = plsc.VectorSubcoreMesh(**_SC_MESH)

    @functools.partial(
        pl.kernel,
        out_type=[jax.ShapeDtypeStruct((1, N_EDGES), f32),
                  jax.ShapeDtypeStruct((NC, WROWS, 128), f32)],
        mesh=mesh,
        scratch_types=[
            pltpu.VMEM((1, CH), i32),
            pltpu.VMEM((1, CH), i32),
            pltpu.VMEM((1, CH), i32),
            pltpu.VMEM((CH, d), f32),
            pltpu.VMEM((CH, d), f32),
            pltpu.VMEM((1, CH), f32),
            pltpu.VMEM((CH, 128), f32),
            pltpu.VMEM_SHARED((WROWS, 128), f32),
            pltpu.SemaphoreType.DMA,
            pltpu.SemaphoreType.DMA,
        ],
        compiler_params=_SC_PARAMS,
    )
    def kern(q_hbm, k_hbm, dst_hbm, src_hbm, z_hbm, w_hbm, wsum_hbm,
             dv, sv, rv, qrows, krows, wbuf, orow2, wsums, sem1, sem2):
        cid = lax.axis_index("c")
        sid = lax.axis_index("s")
        wid = sid * NC + cid
        lane = lax.iota(i32, 16)
        ntile = (NCHUNKS // NW) + jnp.where(wid < NCHUNKS % NW, 1, 0)

        @pl.when(sid < WROWS // 8)
        def _():
            pltpu.sync_copy(z_hbm.at[pl.ds(0, 8)],
                            wsums.at[pl.ds(sid * 8, 8)])
        plsc.subcore_barrier()

        @pl.loop(0, ntile)
        def _chunk(t):
            base = (wid + t * NW) * CH
            pltpu.sync_copy(dst_hbm.at[:, pl.ds(base, CH)], dv)
            pltpu.sync_copy(src_hbm.at[:, pl.ds(base, CH)], sv)
            cq = pltpu.async_copy(q_hbm.at[dv.at[0]], qrows, sem1)
            ck = pltpu.async_copy(k_hbm.at[sv.at[0]], krows, sem2)
            cq.wait()
            ck.wait()

            @pl.loop(0, CH // 16)
            def _grp(eg):
                def edot(j, wv):
                    e = eg * 16 + j
                    acc = qrows[e, pl.ds(0, 16)] * krows[e, pl.ds(0, 16)]
                    for c in range(1, dc):
                        acc = acc + (qrows[e, pl.ds(c * 16, 16)]
                                     * krows[e, pl.ds(c * 16, 16)])
                    return jnp.where(lane == j, jnp.sum(acc), wv)

                wv16 = jnp.exp(
                    pl.loop(0, 16, init_carry=jnp.zeros((16,), f32),
                            unroll=2)(edot) * scale)
                wbuf[0, pl.ds(eg * 16, 16)] = wv16

                dvec = dv[0, pl.ds(eg * 16, 16)]
                rv[0, pl.ds(eg * 16, 16)] = lax.shift_right_logical(dvec, 7)
                cvv = lax.bitwise_and(dvec, 127)
                for j in range(16):
                    e = eg * 16 + j
                    we = wv16[j]
                    ce = cvv[j]
                    for c in range(8):
                        orow2[e, pl.ds(c * 16, 16)] = \
                            jnp.where(lane == ce - c * 16, we, 0.0)

            pltpu.sync_copy(wbuf, w_hbm.at[:, pl.ds(base, CH)])
            pltpu.sync_copy(orow2, wsums.at[rv.at[0]], add=True)

        plsc.subcore_barrier()

        @pl.when(sid < WROWS // 8)
        def _():
            pltpu.sync_copy(wsums.at[pl.ds(sid * 8, 8)],
                            wsum_hbm.at[cid, pl.ds(sid * 8, 8)])

    return kern(q, k, dst, src, z)


def _phase_acc(vtab, dst, src, w, z, split_channels):
    dh = 128
    mesh = plsc.VectorSubcoreMesh(**_SC_MESH)
    nw_eff = NS if split_channels else NW

    @functools.partial(
        pl.kernel,
        out_type=jax.ShapeDtypeStruct((NC, NP, dh), f32),
        mesh=mesh,
        scratch_types=[
            pltpu.VMEM((1, CH), i32),
            pltpu.VMEM((1, CH), i32),
            pltpu.VMEM((1, CH), f32),
            pltpu.VMEM((CH, dh), f32),
            pltpu.VMEM((CH, dh), f32),
            pltpu.VMEM_SHARED((NP, dh), f32),
            pltpu.SemaphoreType.DMA,
        ],
        compiler_params=_SC_PARAMS,
    )
    def kern(v_hbm, dst_hbm, src_hbm, w_hbm, z_hbm, out_hbm,
             sv, dv, wv, vrows, orows, accs, sem):
        cid = lax.axis_index("c")
        sid = lax.axis_index("s")
        wid = sid * NC + cid
        me = sid if split_channels else wid
        ntile = (NCHUNKS // nw_eff) + jnp.where(me < NCHUNKS % nw_eff, 1, 0)

        pltpu.sync_copy(z_hbm.at[pl.ds(sid * RPTB, RPTB)],
                        accs.at[pl.ds(sid * RPTB, RPTB)])
        plsc.subcore_barrier()

        @pl.loop(0, ntile)
        def _chunk(t):
            base = (me + t * nw_eff) * CH
            pltpu.sync_copy(src_hbm.at[:, pl.ds(base, CH)], sv)
            pltpu.sync_copy(dst_hbm.at[:, pl.ds(base, CH)], dv)
            pltpu.sync_copy(w_hbm.at[:, pl.ds(base, CH)], wv)
            if split_channels:
                off = cid * N_NODES
                for u in range(CH // 16):
                    sv[0, pl.ds(u * 16, 16)] = sv[0, pl.ds(u * 16, 16)] + off
            pltpu.async_copy(v_hbm.at[sv.at[0]], vrows, sem).wait()

            @pl.loop(0, CH // 16)
            def _grp(eg):
                wvec = wv[0, pl.ds(eg * 16, 16)]
                for j in range(16):
                    e = eg * 16 + j
                    we = wvec[j]
                    for c in range(dh // 16):
                        orows[e, pl.ds(c * 16, 16)] = \
                            vrows[e, pl.ds(c * 16, 16)] * we

            pltpu.sync_copy(orows, accs.at[dv.at[0]], add=True)

        plsc.subcore_barrier()
        pltpu.sync_copy(accs.at[pl.ds(sid * RPTB, RPTB)],
                        out_hbm.at[cid, pl.ds(sid * RPTB, RPTB)])

    return kern(vtab, dst, src, w, z)


def _wsum_cols(wsum):
    return wsum.reshape(WROWS * 128, 1)[:N_NODES]


@jax.jit
def kernel(features, edge_index, Wq1, bq1, Wk1, bk1, Wv1, bv1, Ws1, bs1,
           Wq2, bq2, Wk2, bk2, Wv2, bv2, Ws2, bs2):
    ei = edge_index.astype(i32)
    src = ei[0].reshape(1, N_EDGES)
    dst = ei[1].reshape(1, N_EDGES)

    d1 = Wq1.shape[0]
    d2 = Wq2.shape[0]

    wcat1 = jnp.concatenate([Wq1.T, Wk1.T, Wv1.T, Ws1.T], axis=1)
    bcat1 = jnp.concatenate([bq1, bk1, bv1, bs1]).reshape(1, 4 * d1)
    wcat2 = jnp.concatenate([Wq2.T, Wk2.T, Wv2.T, Ws2.T], axis=1)
    bcat2 = jnp.concatenate([bq2, bk2, bv2, bs2]).reshape(1, 4 * d2)
    z = jnp.zeros((NP, 128), f32)

    qkvs1 = _proj(features, wcat1, bcat1)
    q1, k1 = qkvs1[:, :d1], qkvs1[:, d1:2 * d1]
    v1, s1 = qkvs1[:, 2 * d1:3 * d1], qkvs1[:, 3 * d1:]
    vcat1 = jnp.concatenate([v1[:, :128], v1[:, 128:]], axis=0)
    w1, wsum1 = _phase_w(q1, k1, dst, src, z, d1)
    acc1 = _phase_acc(vcat1, dst, src, w1, z, split_channels=True)

    qkvs2 = _combine_proj(acc1[0, :N_NODES], acc1[1, :N_NODES],
                          _wsum_cols(wsum1[0]), _wsum_cols(wsum1[1]),
                          s1, wcat2, bcat2, concat_halves=True)

    q2, k2 = qkvs2[:, :d2], qkvs2[:, d2:2 * d2]
    v2, s2 = qkvs2[:, 2 * d2:3 * d2], qkvs2[:, 3 * d2:]
    w2, wsum2 = _phase_w(q2, k2, dst, src, z, d2)
    acc2 = _phase_acc(v2, dst, src, w2, z, split_channels=False)

    return _combine_final(acc2[0, :N_NODES], acc2[1, :N_NODES],
                          _wsum_cols(wsum2[0]), _wsum_cols(wsum2[1]),
                          s2, concat_halves=False)


# speedup vs baseline: 8.4436x; 1.2803x over previous
import functools

import jax
import jax.numpy as jnp
from jax import lax
from jax.experimental import pallas as pl
from jax.experimental.pallas import tpu as pltpu
from jax.experimental.pallas import tpu_sc as plsc

N_NODES = 10000
N_EDGES = 320000
NC = 2
NS = 16
NW = NC * NS
CH = 128
NCHUNKS = N_EDGES // CH
NP = 10240
RPTB = NP // NS
WROWS = 80

_SC_MESH = dict(core_axis_name="c", subcore_axis_name="s", num_cores=NC,
                num_subcores=NS)
_SC_PARAMS = pltpu.CompilerParams(needs_layout_passes=False)

f32 = jnp.float32
i32 = jnp.int32


def _proj(x, w, b, rows=2000):
    m, kdim = x.shape
    nout = w.shape[1]

    def body(x_ref, w_ref, b_ref, o_ref):
        o_ref[...] = jnp.dot(x_ref[...], w_ref[...],
                             preferred_element_type=f32) + b_ref[...]

    return pl.pallas_call(
        body,
        grid=(m // rows,),
        in_specs=[pl.BlockSpec((rows, kdim), lambda i: (i, 0)),
                  pl.BlockSpec((kdim, nout), lambda i: (0, 0)),
                  pl.BlockSpec((1, nout), lambda i: (0, 0))],
        out_specs=pl.BlockSpec((rows, nout), lambda i: (i, 0)),
        out_shape=jax.ShapeDtypeStruct((m, nout), f32),
    )(x, w, b)


def _agg_specs(rows, d, extra):
    return [pl.BlockSpec((rows, 128), lambda i: (i, 0)),
            pl.BlockSpec((rows, 128), lambda i: (i, 0)),
            pl.BlockSpec((rows, 1), lambda i: (i, 0)),
            pl.BlockSpec((rows, 1), lambda i: (i, 0)),
            pl.BlockSpec((rows, d), lambda i: (i, 0))] + extra


def _norm_agg(a0_ref, a1_ref, w0_ref, w1_ref, s_ref, concat_halves):
    if concat_halves:
        agg = jnp.concatenate([a0_ref[...], a1_ref[...]], axis=1)
    else:
        agg = a0_ref[...] + a1_ref[...]
    ws = w0_ref[...] + w1_ref[...]
    return jnp.maximum(agg / (ws + 1e-16) + s_ref[...], 0.0)


def _combine_proj(a0, a1, w0, w1, s, wmat, b, concat_halves, rows=2000):
    m, d = s.shape
    nout = wmat.shape[1]

    def body(a0_ref, a1_ref, w0_ref, w1_ref, s_ref, w_ref, b_ref, o_ref):
        h = _norm_agg(a0_ref, a1_ref, w0_ref, w1_ref, s_ref, concat_halves)
        o_ref[...] = jnp.dot(h, w_ref[...], preferred_element_type=f32) \
            + b_ref[...]

    return pl.pallas_call(
        body,
        grid=(m // rows,),
        in_specs=_agg_specs(rows, d, [
            pl.BlockSpec((d, nout), lambda i: (0, 0)),
            pl.BlockSpec((1, nout), lambda i: (0, 0))]),
        out_specs=pl.BlockSpec((rows, nout), lambda i: (i, 0)),
        out_shape=jax.ShapeDtypeStruct((m, nout), f32),
    )(a0, a1, w0, w1, s, wmat, b)


def _combine_final(a0, a1, w0, w1, s, concat_halves, rows=2000):
    m, d = s.shape

    def body(a0_ref, a1_ref, w0_ref, w1_ref, s_ref, o_ref):
        x = _norm_agg(a0_ref, a1_ref, w0_ref, w1_ref, s_ref, concat_halves)
        mx = jnp.max(x, axis=1, keepdims=True)
        lse = jnp.log(jnp.sum(jnp.exp(x - mx), axis=1, keepdims=True)) + mx
        o_ref[...] = x - lse

    return pl.pallas_call(
        body,
        grid=(m // rows,),
        in_specs=_agg_specs(rows, d, []),
        out_specs=pl.BlockSpec((rows, d), lambda i: (i, 0)),
        out_shape=jax.ShapeDtypeStruct((m, d), f32),
    )(a0, a1, w0, w1, s)


def _phase_w(q, k, dst, src, z, d):
    dc = d // 16
    scale = 1.0 / (d ** 0.5)
    mesh = plsc.VectorSubcoreMesh(**_SC_MESH)

    @functools.partial(
        pl.kernel,
        out_type=[jax.ShapeDtypeStruct((1, N_EDGES), f32),
                  jax.ShapeDtypeStruct((NC, WROWS, 128), f32)],
        mesh=mesh,
        scratch_types=[
            pltpu.VMEM((1, CH), i32),
            pltpu.VMEM((1, CH), i32),
            pltpu.VMEM((1, CH), i32),
            pltpu.VMEM((CH, d), f32),
            pltpu.VMEM((CH, d), f32),
            pltpu.VMEM((1, CH), f32),
            pltpu.VMEM((CH, 128), f32),
            pltpu.VMEM_SHARED((WROWS, 128), f32),
            pltpu.SemaphoreType.DMA,
            pltpu.SemaphoreType.DMA,
        ],
        compiler_params=_SC_PARAMS,
    )
    def kern(q_hbm, k_hbm, dst_hbm, src_hbm, z_hbm, w_hbm, wsum_hbm,
             dv, sv, rv, qrows, krows, wbuf, orow2, wsums, sem1, sem2):
        cid = lax.axis_index("c")
        sid = lax.axis_index("s")
        wid = sid * NC + cid
        lane = lax.iota(i32, 16)
        ntile = (NCHUNKS // NW) + jnp.where(wid < NCHUNKS % NW, 1, 0)

        @pl.when(sid < WROWS // 8)
        def _():
            pltpu.sync_copy(z_hbm.at[pl.ds(0, 8)],
                            wsums.at[pl.ds(sid * 8, 8)])
        plsc.subcore_barrier()

        @pl.loop(0, ntile)
        def _chunk(t):
            base = (wid + t * NW) * CH
            pltpu.sync_copy(dst_hbm.at[:, pl.ds(base, CH)], dv)
            pltpu.sync_copy(src_hbm.at[:, pl.ds(base, CH)], sv)
            cq = pltpu.async_copy(q_hbm.at[dv.at[0]], qrows, sem1)
            ck = pltpu.async_copy(k_hbm.at[sv.at[0]], krows, sem2)
            cq.wait()
            ck.wait()

            @pl.loop(0, CH // 16)
            def _grp(eg):
                def edot(j, wv):
                    e = eg * 16 + j
                    acc = qrows[e, pl.ds(0, 16)] * krows[e, pl.ds(0, 16)]
                    for c in range(1, dc):
                        acc = acc + (qrows[e, pl.ds(c * 16, 16)]
                                     * krows[e, pl.ds(c * 16, 16)])
                    return jnp.where(lane == j, jnp.sum(acc), wv)

                wv16 = jnp.exp(
                    pl.loop(0, 16, init_carry=jnp.zeros((16,), f32),
                            unroll=2)(edot) * scale)
                wbuf[0, pl.ds(eg * 16, 16)] = wv16

                dvec = dv[0, pl.ds(eg * 16, 16)]
                rv[0, pl.ds(eg * 16, 16)] = lax.shift_right_logical(dvec, 7)
                cvv = lax.bitwise_and(dvec, 127)
                for j in range(16):
                    e = eg * 16 + j
                    we = wv16[j]
                    ce = cvv[j]
                    for c in range(8):
                        orow2[e, pl.ds(c * 16, 16)] = \
                            jnp.where(lane == ce - c * 16, we, 0.0)

            pltpu.sync_copy(wbuf, w_hbm.at[:, pl.ds(base, CH)])
            pltpu.sync_copy(orow2, wsums.at[rv.at[0]], add=True)

        plsc.subcore_barrier()

        @pl.when(sid < WROWS // 8)
        def _():
            pltpu.sync_copy(wsums.at[pl.ds(sid * 8, 8)],
                            wsum_hbm.at[cid, pl.ds(sid * 8, 8)])

    return kern(q, k, dst, src, z)


def _phase_acc(vtab, dst, src, w, z, split_channels):
    dh = 128
    mesh = plsc.VectorSubcoreMesh(**_SC_MESH)
    nw_eff = NS if split_channels else NW

    @functools.partial(
        pl.kernel,
        out_type=jax.ShapeDtypeStruct((NC, NP, dh), f32),
        mesh=mesh,
        scratch_types=[
            pltpu.VMEM((1, CH), i32),
            pltpu.VMEM((1, CH), i32),
            pltpu.VMEM((1, CH), i32),
            pltpu.VMEM((1, CH), i32),
            pltpu.VMEM((1, CH), f32),
            pltpu.VMEM((1, CH), f32),
            pltpu.VMEM((CH, dh), f32),
            pltpu.VMEM((CH, dh), f32),
            pltpu.VMEM_SHARED((NP, dh), f32),
            pltpu.SemaphoreType.DMA,
            pltpu.SemaphoreType.DMA,
            pltpu.SemaphoreType.DMA,
            pltpu.SemaphoreType.DMA,
        ],
        compiler_params=_SC_PARAMS,
    )
    def kern(v_hbm, dst_hbm, src_hbm, w_hbm, z_hbm, out_hbm,
             svA, svB, dvA, dvB, wvA, wvB, vrA, vrB, accs,
             gA, gB, cA, cB):
        cid = lax.axis_index("c")
        sid = lax.axis_index("s")
        wid = sid * NC + cid
        me = sid if split_channels else wid
        ntile = (NCHUNKS // nw_eff) + jnp.where(me < NCHUNKS % nw_eff, 1, 0)
        tp = ntile // 2

        pltpu.sync_copy(z_hbm.at[pl.ds(sid * RPTB, RPTB)],
                        accs.at[pl.ds(sid * RPTB, RPTB)])
        plsc.subcore_barrier()

        def issue(t, sv, dv, wv, vr, gsem):
            base = (me + t * nw_eff) * CH
            pltpu.sync_copy(src_hbm.at[:, pl.ds(base, CH)], sv)
            pltpu.sync_copy(dst_hbm.at[:, pl.ds(base, CH)], dv)
            pltpu.sync_copy(w_hbm.at[:, pl.ds(base, CH)], wv)
            if split_channels:
                off = cid * N_NODES
                for u in range(CH // 16):
                    sv[0, pl.ds(u * 16, 16)] = sv[0, pl.ds(u * 16, 16)] + off
            pltpu.async_copy(v_hbm.at[sv.at[0]], vr, gsem)

        def wait_gather(sv, vr, gsem):
            pltpu.make_async_copy(v_hbm.at[sv.at[0]], vr, gsem).wait()

        def compute(vr, wv):
            @pl.loop(0, CH // 16)
            def _grp(eg):
                wvec = wv[0, pl.ds(eg * 16, 16)]
                for j in range(16):
                    e = eg * 16 + j
                    we = wvec[j]
                    for c in range(dh // 16):
                        vr[e, pl.ds(c * 16, 16)] = \
                            vr[e, pl.ds(c * 16, 16)] * we

        def wait_scatter(dv, vr, csem):
            pltpu.make_async_copy(vr, accs.at[dv.at[0]], csem).wait()

        @pl.when(tp > 0)
        def _():
            issue(0, svA, dvA, wvA, vrA, gA)

        @pl.loop(0, tp)
        def _pipe(p):
            t0 = 2 * p
            issue(t0 + 1, svB, dvB, wvB, vrB, gB)
            wait_gather(svA, vrA, gA)
            compute(vrA, wvA)
            pltpu.async_copy(vrA, accs.at[dvA.at[0]], cA, add=True)
            wait_gather(svB, vrB, gB)
            compute(vrB, wvB)
            pltpu.async_copy(vrB, accs.at[dvB.at[0]], cB, add=True)

            @pl.when(p < tp - 1)
            def _():
                wait_scatter(dvA, vrA, cA)
                issue(t0 + 2, svA, dvA, wvA, vrA, gA)
            wait_scatter(dvB, vrB, cB)

        @pl.when(tp > 0)
        def _():
            wait_scatter(dvA, vrA, cA)

        @pl.when(ntile % 2 == 1)
        def _():
            issue(ntile - 1, svA, dvA, wvA, vrA, gA)
            wait_gather(svA, vrA, gA)
            compute(vrA, wvA)
            pltpu.async_copy(vrA, accs.at[dvA.at[0]], cA, add=True)
            wait_scatter(dvA, vrA, cA)

        plsc.subcore_barrier()
        pltpu.sync_copy(accs.at[pl.ds(sid * RPTB, RPTB)],
                        out_hbm.at[cid, pl.ds(sid * RPTB, RPTB)])

    return kern(vtab, dst, src, w, z)


def _wsum_cols(wsum):
    return wsum.reshape(WROWS * 128, 1)[:N_NODES]


@jax.jit
def kernel(features, edge_index, Wq1, bq1, Wk1, bk1, Wv1, bv1, Ws1, bs1,
           Wq2, bq2, Wk2, bk2, Wv2, bv2, Ws2, bs2):
    ei = edge_index.astype(i32)
    src = ei[0].reshape(1, N_EDGES)
    dst = ei[1].reshape(1, N_EDGES)

    d1 = Wq1.shape[0]
    d2 = Wq2.shape[0]

    wcat1 = jnp.concatenate([Wq1.T, Wk1.T, Wv1.T, Ws1.T], axis=1)
    bcat1 = jnp.concatenate([bq1, bk1, bv1, bs1]).reshape(1, 4 * d1)
    wcat2 = jnp.concatenate([Wq2.T, Wk2.T, Wv2.T, Ws2.T], axis=1)
    bcat2 = jnp.concatenate([bq2, bk2, bv2, bs2]).reshape(1, 4 * d2)
    z = jnp.zeros((NP, 128), f32)

    qkvs1 = _proj(features, wcat1, bcat1)
    q1, k1 = qkvs1[:, :d1], qkvs1[:, d1:2 * d1]
    v1, s1 = qkvs1[:, 2 * d1:3 * d1], qkvs1[:, 3 * d1:]
    vcat1 = jnp.concatenate([v1[:, :128], v1[:, 128:]], axis=0)
    w1, wsum1 = _phase_w(q1, k1, dst, src, z, d1)
    acc1 = _phase_acc(vcat1, dst, src, w1, z, split_channels=True)

    qkvs2 = _combine_proj(acc1[0, :N_NODES], acc1[1, :N_NODES],
                          _wsum_cols(wsum1[0]), _wsum_cols(wsum1[1]),
                          s1, wcat2, bcat2, concat_halves=True)

    q2, k2 = qkvs2[:, :d2], qkvs2[:, d2:2 * d2]
    v2, s2 = qkvs2[:, 2 * d2:3 * d2], qkvs2[:, 3 * d2:]
    w2, wsum2 = _phase_w(q2, k2, dst, src, z, d2)
    acc2 = _phase_acc(v2, dst, src, w2, z, split_channels=False)

    return _combine_final(acc2[0, :N_NODES], acc2[1, :N_NODES],
                          _wsum_cols(wsum2[0]), _wsum_cols(wsum2[1]),
                          s2, concat_halves=False)


# speedup vs baseline: 8.9359x; 1.0583x over previous
import functools

import jax
import jax.numpy as jnp
from jax import lax
from jax.experimental import pallas as pl
from jax.experimental.pallas import tpu as pltpu
from jax.experimental.pallas import tpu_sc as plsc

N_NODES = 10000
N_EDGES = 320000
NC = 2
NS = 16
NW = NC * NS
CH = 128
NCHUNKS = N_EDGES // CH
NP = 10240
RPTB = NP // NS
WROWS = 80

_SC_MESH = dict(core_axis_name="c", subcore_axis_name="s", num_cores=NC,
                num_subcores=NS)
_SC_PARAMS = pltpu.CompilerParams(needs_layout_passes=False)

f32 = jnp.float32
i32 = jnp.int32


def _proj(x, w, b, rows=2000):
    m, kdim = x.shape
    nout = w.shape[1]

    def body(x_ref, w_ref, b_ref, o_ref):
        o_ref[...] = jnp.dot(x_ref[...], w_ref[...],
                             preferred_element_type=f32) + b_ref[...]

    return pl.pallas_call(
        body,
        grid=(m // rows,),
        in_specs=[pl.BlockSpec((rows, kdim), lambda i: (i, 0)),
                  pl.BlockSpec((kdim, nout), lambda i: (0, 0)),
                  pl.BlockSpec((1, nout), lambda i: (0, 0))],
        out_specs=pl.BlockSpec((rows, nout), lambda i: (i, 0)),
        out_shape=jax.ShapeDtypeStruct((m, nout), f32),
    )(x, w, b)


def _agg_specs(rows, d, extra):
    return [pl.BlockSpec((rows, 128), lambda i: (i, 0)),
            pl.BlockSpec((rows, 128), lambda i: (i, 0)),
            pl.BlockSpec((rows, 1), lambda i: (i, 0)),
            pl.BlockSpec((rows, 1), lambda i: (i, 0)),
            pl.BlockSpec((rows, d), lambda i: (i, 0))] + extra


def _norm_agg(a0_ref, a1_ref, w0_ref, w1_ref, s_ref, concat_halves):
    if concat_halves:
        agg = jnp.concatenate([a0_ref[...], a1_ref[...]], axis=1)
    else:
        agg = a0_ref[...] + a1_ref[...]
    ws = w0_ref[...] + w1_ref[...]
    return jnp.maximum(agg / (ws + 1e-16) + s_ref[...], 0.0)


def _combine_proj(a0, a1, w0, w1, s, wmat, b, concat_halves, rows=2000):
    m, d = s.shape
    nout = wmat.shape[1]

    def body(a0_ref, a1_ref, w0_ref, w1_ref, s_ref, w_ref, b_ref, o_ref):
        h = _norm_agg(a0_ref, a1_ref, w0_ref, w1_ref, s_ref, concat_halves)
        o_ref[...] = jnp.dot(h, w_ref[...], preferred_element_type=f32) \
            + b_ref[...]

    return pl.pallas_call(
        body,
        grid=(m // rows,),
        in_specs=_agg_specs(rows, d, [
            pl.BlockSpec((d, nout), lambda i: (0, 0)),
            pl.BlockSpec((1, nout), lambda i: (0, 0))]),
        out_specs=pl.BlockSpec((rows, nout), lambda i: (i, 0)),
        out_shape=jax.ShapeDtypeStruct((m, nout), f32),
    )(a0, a1, w0, w1, s, wmat, b)


def _combine_final(a0, a1, w0, w1, s, concat_halves, rows=2000):
    m, d = s.shape

    def body(a0_ref, a1_ref, w0_ref, w1_ref, s_ref, o_ref):
        x = _norm_agg(a0_ref, a1_ref, w0_ref, w1_ref, s_ref, concat_halves)
        mx = jnp.max(x, axis=1, keepdims=True)
        lse = jnp.log(jnp.sum(jnp.exp(x - mx), axis=1, keepdims=True)) + mx
        o_ref[...] = x - lse

    return pl.pallas_call(
        body,
        grid=(m // rows,),
        in_specs=_agg_specs(rows, d, []),
        out_specs=pl.BlockSpec((rows, d), lambda i: (i, 0)),
        out_shape=jax.ShapeDtypeStruct((m, d), f32),
    )(a0, a1, w0, w1, s)


def _phase_w(q, k, dst, src, z, d):
    dc = d // 16
    scale = 1.0 / (d ** 0.5)
    mesh = plsc.VectorSubcoreMesh(**_SC_MESH)

    @functools.partial(
        pl.kernel,
        out_type=[jax.ShapeDtypeStruct((1, N_EDGES), f32),
                  jax.ShapeDtypeStruct((NC, WROWS, 128), f32)],
        mesh=mesh,
        scratch_types=[
            pltpu.VMEM((1, CH), i32),
            pltpu.VMEM((1, CH), i32),
            pltpu.VMEM((1, CH), i32),
            pltpu.VMEM((1, CH), i32),
            pltpu.VMEM((CH, d), f32),
            pltpu.VMEM((CH, d), f32),
            pltpu.VMEM((1, CH), f32),
            pltpu.VMEM((1, CH), f32),
            pltpu.VMEM((CH, 128), f32),
            pltpu.VMEM((CH, 128), f32),
            pltpu.VMEM_SHARED((WROWS, 128), f32),
            pltpu.SemaphoreType.DMA,
            pltpu.SemaphoreType.DMA,
            pltpu.SemaphoreType.DMA,
            pltpu.SemaphoreType.DMA,
            pltpu.SemaphoreType.DMA,
            pltpu.SemaphoreType.DMA,
        ],
        compiler_params=_SC_PARAMS,
    )
    def kern(q_hbm, k_hbm, dst_hbm, src_hbm, z_hbm, w_hbm, wsum_hbm,
             dv, sv, rvA, rvB, qrows, krows, wbA, wbB, o2A, o2B, wsums,
             sq, sk, cA, cB, wA, wB):
        cid = lax.axis_index("c")
        sid = lax.axis_index("s")
        wid = sid * NC + cid
        lane = lax.iota(i32, 16)
        ntile = (NCHUNKS // NW) + jnp.where(wid < NCHUNKS % NW, 1, 0)
        tp = ntile // 2

        @pl.when(sid < WROWS // 8)
        def _():
            pltpu.sync_copy(z_hbm.at[pl.ds(0, 8)],
                            wsums.at[pl.ds(sid * 8, 8)])
        plsc.subcore_barrier()

        def front(t, rv, wbuf, o2, csem, wsem):
            base = (wid + t * NW) * CH
            pltpu.sync_copy(dst_hbm.at[:, pl.ds(base, CH)], dv)
            pltpu.sync_copy(src_hbm.at[:, pl.ds(base, CH)], sv)
            cq = pltpu.async_copy(q_hbm.at[dv.at[0]], qrows, sq)
            ck = pltpu.async_copy(k_hbm.at[sv.at[0]], krows, sk)
            cq.wait()
            ck.wait()

            @pl.loop(0, CH // 16)
            def _grp(eg):
                def edot(j, wv):
                    e = eg * 16 + j
                    acc = qrows[e, pl.ds(0, 16)] * krows[e, pl.ds(0, 16)]
                    for c in range(1, dc):
                        acc = acc + (qrows[e, pl.ds(c * 16, 16)]
                                     * krows[e, pl.ds(c * 16, 16)])
                    return jnp.where(lane == j, jnp.sum(acc), wv)

                wv16 = jnp.exp(
                    pl.loop(0, 16, init_carry=jnp.zeros((16,), f32),
                            unroll=2)(edot) * scale)
                wbuf[0, pl.ds(eg * 16, 16)] = wv16

                dvec = dv[0, pl.ds(eg * 16, 16)]
                rv[0, pl.ds(eg * 16, 16)] = lax.shift_right_logical(dvec, 7)
                cvv = lax.bitwise_and(dvec, 127)
                for j in range(16):
                    e = eg * 16 + j
                    we = wv16[j]
                    ce = cvv[j]
                    for c in range(8):
                        o2[e, pl.ds(c * 16, 16)] = \
                            jnp.where(lane == ce - c * 16, we, 0.0)

            pltpu.async_copy(o2, wsums.at[rv.at[0]], csem, add=True)
            pltpu.async_copy(wbuf, w_hbm.at[:, pl.ds(base, CH)], wsem)

        def drain(rv, wbuf, o2, csem, wsem):
            pltpu.make_async_copy(o2, wsums.at[rv.at[0]], csem).wait()
            pltpu.make_async_copy(wbuf, w_hbm.at[:, pl.ds(0, CH)],
                                  wsem).wait()

        @pl.loop(0, tp)
        def _pipe(p):
            t0 = 2 * p

            @pl.when(p > 0)
            def _():
                drain(rvA, wbA, o2A, cA, wA)
            front(t0, rvA, wbA, o2A, cA, wA)

            @pl.when(p > 0)
            def _():
                drain(rvB, wbB, o2B, cB, wB)
            front(t0 + 1, rvB, wbB, o2B, cB, wB)

        @pl.when(tp > 0)
        def _():
            drain(rvA, wbA, o2A, cA, wA)
            drain(rvB, wbB, o2B, cB, wB)

        @pl.when(ntile % 2 == 1)
        def _():
            front(ntile - 1, rvA, wbA, o2A, cA, wA)
            drain(rvA, wbA, o2A, cA, wA)

        plsc.subcore_barrier()

        @pl.when(sid < WROWS // 8)
        def _():
            pltpu.sync_copy(wsums.at[pl.ds(sid * 8, 8)],
                            wsum_hbm.at[cid, pl.ds(sid * 8, 8)])

    return kern(q, k, dst, src, z)


def _phase_acc(vtab, dst, src, w, z, split_channels):
    dh = 128
    mesh = plsc.VectorSubcoreMesh(**_SC_MESH)
    nw_eff = NS if split_channels else NW

    @functools.partial(
        pl.kernel,
        out_type=jax.ShapeDtypeStruct((NC, NP, dh), f32),
        mesh=mesh,
        scratch_types=[
            pltpu.VMEM((1, CH), i32),
            pltpu.VMEM((1, CH), i32),
            pltpu.VMEM((1, CH), i32),
            pltpu.VMEM((1, CH), i32),
            pltpu.VMEM((1, CH), f32),
            pltpu.VMEM((1, CH), f32),
            pltpu.VMEM((CH, dh), f32),
            pltpu.VMEM((CH, dh), f32),
            pltpu.VMEM_SHARED((NP, dh), f32),
            pltpu.SemaphoreType.DMA,
            pltpu.SemaphoreType.DMA,
            pltpu.SemaphoreType.DMA,
            pltpu.SemaphoreType.DMA,
        ],
        compiler_params=_SC_PARAMS,
    )
    def kern(v_hbm, dst_hbm, src_hbm, w_hbm, z_hbm, out_hbm,
             svA, svB, dvA, dvB, wvA, wvB, vrA, vrB, accs,
             gA, gB, cA, cB):
        cid = lax.axis_index("c")
        sid = lax.axis_index("s")
        wid = sid * NC + cid
        me = sid if split_channels else wid
        ntile = (NCHUNKS // nw_eff) + jnp.where(me < NCHUNKS % nw_eff, 1, 0)
        tp = ntile // 2

        pltpu.sync_copy(z_hbm.at[pl.ds(sid * RPTB, RPTB)],
                        accs.at[pl.ds(sid * RPTB, RPTB)])
        plsc.subcore_barrier()

        def issue(t, sv, dv, wv, vr, gsem):
            base = (me + t * nw_eff) * CH
            pltpu.sync_copy(src_hbm.at[:, pl.ds(base, CH)], sv)
            pltpu.sync_copy(dst_hbm.at[:, pl.ds(base, CH)], dv)
            pltpu.sync_copy(w_hbm.at[:, pl.ds(base, CH)], wv)
            if split_channels:
                off = cid * N_NODES
                for u in range(CH // 16):
                    sv[0, pl.ds(u * 16, 16)] = sv[0, pl.ds(u * 16, 16)] + off
            pltpu.async_copy(v_hbm.at[sv.at[0]], vr, gsem)

        def wait_gather(sv, vr, gsem):
            pltpu.make_async_copy(v_hbm.at[sv.at[0]], vr, gsem).wait()

        def compute(vr, wv):
            @pl.loop(0, CH // 16)
            def _grp(eg):
                wvec = wv[0, pl.ds(eg * 16, 16)]
                for j in range(16):
                    e = eg * 16 + j
                    we = wvec[j]
                    for c in range(dh // 16):
                        vr[e, pl.ds(c * 16, 16)] = \
                            vr[e, pl.ds(c * 16, 16)] * we

        def wait_scatter(dv, vr, csem):
            pltpu.make_async_copy(vr, accs.at[dv.at[0]], csem).wait()

        @pl.when(tp > 0)
        def _():
            issue(0, svA, dvA, wvA, vrA, gA)

        @pl.loop(0, tp)
        def _pipe(p):
            t0 = 2 * p
            issue(t0 + 1, svB, dvB, wvB, vrB, gB)
            wait_gather(svA, vrA, gA)
            compute(vrA, wvA)
            pltpu.async_copy(vrA, accs.at[dvA.at[0]], cA, add=True)
            wait_gather(svB, vrB, gB)
            compute(vrB, wvB)
            pltpu.async_copy(vrB, accs.at[dvB.at[0]], cB, add=True)

            @pl.when(p < tp - 1)
            def _():
                wait_scatter(dvA, vrA, cA)
                issue(t0 + 2, svA, dvA, wvA, vrA, gA)
            wait_scatter(dvB, vrB, cB)

        @pl.when(tp > 0)
        def _():
            wait_scatter(dvA, vrA, cA)

        @pl.when(ntile % 2 == 1)
        def _():
            issue(ntile - 1, svA, dvA, wvA, vrA, gA)
            wait_gather(svA, vrA, gA)
            compute(vrA, wvA)
            pltpu.async_copy(vrA, accs.at[dvA.at[0]], cA, add=True)
            wait_scatter(dvA, vrA, cA)

        plsc.subcore_barrier()
        pltpu.sync_copy(accs.at[pl.ds(sid * RPTB, RPTB)],
                        out_hbm.at[cid, pl.ds(sid * RPTB, RPTB)])

    return kern(vtab, dst, src, w, z)


def _wsum_cols(wsum):
    return wsum.reshape(WROWS * 128, 1)[:N_NODES]


@jax.jit
def kernel(features, edge_index, Wq1, bq1, Wk1, bk1, Wv1, bv1, Ws1, bs1,
           Wq2, bq2, Wk2, bk2, Wv2, bv2, Ws2, bs2):
    ei = edge_index.astype(i32)
    src = ei[0].reshape(1, N_EDGES)
    dst = ei[1].reshape(1, N_EDGES)

    d1 = Wq1.shape[0]
    d2 = Wq2.shape[0]

    wcat1 = jnp.concatenate([Wq1.T, Wk1.T, Wv1.T, Ws1.T], axis=1)
    bcat1 = jnp.concatenate([bq1, bk1, bv1, bs1]).reshape(1, 4 * d1)
    wcat2 = jnp.concatenate([Wq2.T, Wk2.T, Wv2.T, Ws2.T], axis=1)
    bcat2 = jnp.concatenate([bq2, bk2, bv2, bs2]).reshape(1, 4 * d2)
    z = jnp.zeros((NP, 128), f32)

    qkvs1 = _proj(features, wcat1, bcat1)
    q1, k1 = qkvs1[:, :d1], qkvs1[:, d1:2 * d1]
    v1, s1 = qkvs1[:, 2 * d1:3 * d1], qkvs1[:, 3 * d1:]
    vcat1 = jnp.concatenate([v1[:, :128], v1[:, 128:]], axis=0)
    w1, wsum1 = _phase_w(q1, k1, dst, src, z, d1)
    acc1 = _phase_acc(vcat1, dst, src, w1, z, split_channels=True)

    qkvs2 = _combine_proj(acc1[0, :N_NODES], acc1[1, :N_NODES],
                          _wsum_cols(wsum1[0]), _wsum_cols(wsum1[1]),
                          s1, wcat2, bcat2, concat_halves=True)

    q2, k2 = qkvs2[:, :d2], qkvs2[:, d2:2 * d2]
    v2, s2 = qkvs2[:, 2 * d2:3 * d2], qkvs2[:, 3 * d2:]
    w2, wsum2 = _phase_w(q2, k2, dst, src, z, d2)
    acc2 = _phase_acc(v2, dst, src, w2, z, split_channels=False)

    return _combine_final(acc2[0, :N_NODES], acc2[1, :N_NODES],
                          _wsum_cols(wsum2[0]), _wsum_cols(wsum2[1]),
                          s2, concat_halves=False)
